# baseline (speedup 1.0000x reference)
import functools

import jax
import jax.numpy as jnp
from jax import lax
from jax.experimental import pallas as pl
from jax.experimental.pallas import tpu as pltpu
from jax.experimental.pallas import tpu_sc as plsc

H = 768
EPS = 1e-12

NC, NS = 2, 16
NW = NC * NS
CHUNK = 16
NBUF = 8

PIECES = (8192, 8192)
BT = 1024


def _sc_gather(table, idx_full, start, piece):
    rows_per_w = piece // NW
    n_chunks = rows_per_w // CHUNK

    mesh = plsc.VectorSubcoreMesh(core_axis_name="c", subcore_axis_name="s")

    @functools.partial(
        pl.kernel,
        out_type=jax.ShapeDtypeStruct((piece, H), jnp.float32),
        mesh=mesh,
        scratch_types=(
            [pltpu.VMEM((rows_per_w,), jnp.int32)]
            + [pltpu.VMEM((CHUNK, H), jnp.float32) for _ in range(NBUF)]
            + [pltpu.SemaphoreType.DMA for _ in range(2 * NBUF)]
        ),
    )
    def gather_kernel(table_hbm, idx_hbm, out_hbm, idx_v, *bufs_and_sems):
        rows = bufs_and_sems[:NBUF]
        gsem = bufs_and_sems[NBUF:2 * NBUF]
        wsem = bufs_and_sems[2 * NBUF:3 * NBUF]
        wid = lax.axis_index("s") * NC + lax.axis_index("c")
        base = wid * rows_per_w
        pltpu.sync_copy(idx_hbm.at[pl.ds(start + base, rows_per_w)], idx_v)

        def gather(c):
            b = c % NBUF
            return pltpu.async_copy(
                table_hbm.at[idx_v.at[pl.ds(c * CHUNK, CHUNK)]],
                rows[b], gsem[b])

        gcp = [None] * n_chunks
        wcp = [None] * n_chunks
        for k in range(min(NBUF - 1, n_chunks)):
            gcp[k] = gather(k)
        for c in range(n_chunks):
            b = c % NBUF
            gcp[c].wait()
            nxt = c + NBUF - 1
            if nxt < n_chunks:
                if c >= 1:
                    wcp[c - 1].wait()
                gcp[nxt] = gather(nxt)
            wcp[c] = pltpu.async_copy(
                rows[b], out_hbm.at[pl.ds(base + c * CHUNK, CHUNK)], wsem[b])
        for c in range(max(0, n_chunks - NBUF), n_chunks):
            wcp[c].wait()

    return gather_kernel(table, idx_full)


def _tc_body(d_ref, g_ref, t_ref, w_ref, b_ref, gam_ref, bet_ref, o_ref):
    ttl = lax.dot_general(t_ref[...], w_ref[...],
                          (((0,), (0,)), ((), ())),
                          preferred_element_type=jnp.float32,
                          precision=lax.Precision.DEFAULT)
    x = g_ref[...] + ttl + b_ref[...]
    mu = jnp.mean(x, axis=-1, keepdims=True)
    d = x - mu
    var = jnp.mean(d * d, axis=-1, keepdims=True)
    o_ref[...] = d * lax.rsqrt(var + EPS) * gam_ref[...] + bet_ref[...]


def _tc_fused_piece(dst, gathered, tt, tt_w, tt_b, gamma, beta, blk_off):
    n_tok = tt.shape[1]
    piece = gathered.shape[0]

    def body(*refs):
        if dst is None:
            _tc_body(None, *refs)
        else:
            _tc_body(*refs)

    dst_in_specs = [] if dst is None else [pl.BlockSpec((8, 128), lambda i: (0, 0))]
    dst_args = () if dst is None else (dst,)

    return pl.pallas_call(
        body,
        grid=(piece // BT,),
        in_specs=dst_in_specs + [
            pl.BlockSpec((BT, H), lambda i: (i, 0)),
            pl.BlockSpec((3, BT), lambda i, o=blk_off: (0, i + o)),
            pl.BlockSpec((3, H), lambda i: (0, 0)),
            pl.BlockSpec((1, H), lambda i: (0, 0)),
            pl.BlockSpec((1, H), lambda i: (0, 0)),
            pl.BlockSpec((1, H), lambda i: (0, 0)),
        ],
        out_specs=pl.BlockSpec((BT, H), lambda i, o=blk_off: (i + o, 0)),
        out_shape=jax.ShapeDtypeStruct((n_tok, H), jnp.float32),
        input_output_aliases={} if dst is None else {0: 0},
    )(*dst_args, gathered, tt, tt_w, tt_b, gamma, beta)


@jax.jit
def kernel(input_ids, token_type_ids, word_table, tt_w, tt_b, ln_gamma, ln_beta):
    b, s = input_ids.shape
    idx = input_ids.reshape(-1).astype(jnp.int32)
    tt = token_type_ids.reshape(-1, 3).T
    tt_b2 = tt_b.reshape(1, H)
    gam2 = ln_gamma.reshape(1, H)
    bet2 = ln_beta.reshape(1, H)

    out = None
    start = 0
    for piece in PIECES:
        gathered = _sc_gather(word_table, idx, start, piece)
        out = _tc_fused_piece(out, gathered, tt, tt_w, tt_b2, gam2, bet2,
                              start // BT)
        start += piece
    return out.reshape(b, s, H)


# speedup vs baseline: 1.0097x; 1.0097x over previous
import functools

import jax
import jax.numpy as jnp
from jax import lax
from jax.experimental import pallas as pl
from jax.experimental.pallas import tpu as pltpu
from jax.experimental.pallas import tpu_sc as plsc

H = 768
EPS = 1e-12

NC, NS = 2, 16
NW = NC * NS
CHUNK = 32
NBUF = 4

PIECES = (8192, 8192)
BT = 2048


def _sc_gather(table, idx_full, start, piece):
    rows_per_w = piece // NW
    n_chunks = rows_per_w // CHUNK

    mesh = plsc.VectorSubcoreMesh(core_axis_name="c", subcore_axis_name="s")

    @functools.partial(
        pl.kernel,
        out_type=jax.ShapeDtypeStruct((piece, H), jnp.float32),
        mesh=mesh,
        scratch_types=(
            [pltpu.VMEM((rows_per_w,), jnp.int32)]
            + [pltpu.VMEM((CHUNK, H), jnp.float32) for _ in range(NBUF)]
            + [pltpu.SemaphoreType.DMA for _ in range(2 * NBUF)]
        ),
    )
    def gather_kernel(table_hbm, idx_hbm, out_hbm, idx_v, *bufs_and_sems):
        rows = bufs_and_sems[:NBUF]
        gsem = bufs_and_sems[NBUF:2 * NBUF]
        wsem = bufs_and_sems[2 * NBUF:3 * NBUF]
        wid = lax.axis_index("s") * NC + lax.axis_index("c")
        base = wid * rows_per_w
        pltpu.sync_copy(idx_hbm.at[pl.ds(start + base, rows_per_w)], idx_v)

        def gather(c):
            b = c % NBUF
            return pltpu.async_copy(
                table_hbm.at[idx_v.at[pl.ds(c * CHUNK, CHUNK)]],
                rows[b], gsem[b])

        gcp = [None] * n_chunks
        wcp = [None] * n_chunks
        for k in range(min(NBUF - 1, n_chunks)):
            gcp[k] = gather(k)
        for c in range(n_chunks):
            b = c % NBUF
            gcp[c].wait()
            nxt = c + NBUF - 1
            if nxt < n_chunks:
                if c >= 1:
                    wcp[c - 1].wait()
                gcp[nxt] = gather(nxt)
            wcp[c] = pltpu.async_copy(
                rows[b], out_hbm.at[pl.ds(base + c * CHUNK, CHUNK)], wsem[b])
        for c in range(max(0, n_chunks - NBUF), n_chunks):
            wcp[c].wait()

    return gather_kernel(table, idx_full)


def _tc_body(d_ref, g_ref, t_ref, w_ref, b_ref, gam_ref, bet_ref, o_ref):
    ttl = lax.dot_general(t_ref[...], w_ref[...],
                          (((0,), (0,)), ((), ())),
                          preferred_element_type=jnp.float32,
                          precision=lax.Precision.DEFAULT)
    x = g_ref[...] + ttl + b_ref[...]
    mu = jnp.mean(x, axis=-1, keepdims=True)
    d = x - mu
    var = jnp.mean(d * d, axis=-1, keepdims=True)
    o_ref[...] = d * lax.rsqrt(var + EPS) * gam_ref[...] + bet_ref[...]


def _tc_fused_piece(dst, gathered, tt, tt_w, tt_b, gamma, beta, blk_off):
    n_tok = tt.shape[1]
    piece = gathered.shape[0]

    def body(*refs):
        if dst is None:
            _tc_body(None, *refs)
        else:
            _tc_body(*refs)

    dst_in_specs = [] if dst is None else [pl.BlockSpec((8, 128), lambda i: (0, 0))]
    dst_args = () if dst is None else (dst,)

    return pl.pallas_call(
        body,
        grid=(piece // BT,),
        in_specs=dst_in_specs + [
            pl.BlockSpec((BT, H), lambda i: (i, 0)),
            pl.BlockSpec((3, BT), lambda i, o=blk_off: (0, i + o)),
            pl.BlockSpec((3, H), lambda i: (0, 0)),
            pl.BlockSpec((1, H), lambda i: (0, 0)),
            pl.BlockSpec((1, H), lambda i: (0, 0)),
            pl.BlockSpec((1, H), lambda i: (0, 0)),
        ],
        out_specs=pl.BlockSpec((BT, H), lambda i, o=blk_off: (i + o, 0)),
        out_shape=jax.ShapeDtypeStruct((n_tok, H), jnp.float32),
        input_output_aliases={} if dst is None else {0: 0},
    )(*dst_args, gathered, tt, tt_w, tt_b, gamma, beta)


@jax.jit
def kernel(input_ids, token_type_ids, word_table, tt_w, tt_b, ln_gamma, ln_beta):
    b, s = input_ids.shape
    idx = input_ids.reshape(-1).astype(jnp.int32)
    tt = token_type_ids.reshape(-1, 3).T
    tt_b2 = tt_b.reshape(1, H)
    gam2 = ln_gamma.reshape(1, H)
    bet2 = ln_beta.reshape(1, H)

    out = None
    start = 0
    for piece in PIECES:
        gathered = _sc_gather(word_table, idx, start, piece)
        out = _tc_fused_piece(out, gathered, tt, tt_w, tt_b2, gam2, bet2,
                              start // BT)
        start += piece
    return out.reshape(b, s, H)


# speedup vs baseline: 1.0181x; 1.0083x over previous
import functools

import jax
import jax.numpy as jnp
from jax import lax
from jax.experimental import pallas as pl
from jax.experimental.pallas import tpu as pltpu
from jax.experimental.pallas import tpu_sc as plsc

H = 768
EPS = 1e-12

NC, NS = 2, 16
NW = NC * NS
CHUNK = 32
NBUF = 4

PIECES = (8192, 8192)
BT = 4096


def _sc_gather(table, idx_full, start, piece):
    rows_per_w = piece // NW
    n_chunks = rows_per_w // CHUNK

    mesh = plsc.VectorSubcoreMesh(core_axis_name="c", subcore_axis_name="s")

    @functools.partial(
        pl.kernel,
        out_type=jax.ShapeDtypeStruct((piece, H), jnp.float32),
        mesh=mesh,
        scratch_types=(
            [pltpu.VMEM((rows_per_w,), jnp.int32)]
            + [pltpu.VMEM((CHUNK, H), jnp.float32) for _ in range(NBUF)]
            + [pltpu.SemaphoreType.DMA for _ in range(2 * NBUF)]
        ),
    )
    def gather_kernel(table_hbm, idx_hbm, out_hbm, idx_v, *bufs_and_sems):
        rows = bufs_and_sems[:NBUF]
        gsem = bufs_and_sems[NBUF:2 * NBUF]
        wsem = bufs_and_sems[2 * NBUF:3 * NBUF]
        wid = lax.axis_index("s") * NC + lax.axis_index("c")
        base = wid * rows_per_w
        pltpu.sync_copy(idx_hbm.at[pl.ds(start + base, rows_per_w)], idx_v)

        def gather(c):
            b = c % NBUF
            return pltpu.async_copy(
                table_hbm.at[idx_v.at[pl.ds(c * CHUNK, CHUNK)]],
                rows[b], gsem[b])

        gcp = [None] * n_chunks
        wcp = [None] * n_chunks
        for k in range(min(NBUF - 1, n_chunks)):
            gcp[k] = gather(k)
        for c in range(n_chunks):
            b = c % NBUF
            gcp[c].wait()
            nxt = c + NBUF - 1
            if nxt < n_chunks:
                if c >= 1:
                    wcp[c - 1].wait()
                gcp[nxt] = gather(nxt)
            wcp[c] = pltpu.async_copy(
                rows[b], out_hbm.at[pl.ds(base + c * CHUNK, CHUNK)], wsem[b])
        for c in range(max(0, n_chunks - NBUF), n_chunks):
            wcp[c].wait()

    return gather_kernel(table, idx_full)


def _tc_body(d_ref, g_ref, t_ref, w_ref, b_ref, gam_ref, bet_ref, o_ref):
    ttl = lax.dot_general(t_ref[...], w_ref[...],
                          (((0,), (0,)), ((), ())),
                          preferred_element_type=jnp.float32,
                          precision=lax.Precision.DEFAULT)
    x = g_ref[...] + ttl + b_ref[...]
    mu = jnp.mean(x, axis=-1, keepdims=True)
    d = x - mu
    var = jnp.mean(d * d, axis=-1, keepdims=True)
    o_ref[...] = d * lax.rsqrt(var + EPS) * gam_ref[...] + bet_ref[...]


def _tc_fused_piece(dst, gathered, tt, tt_w, tt_b, gamma, beta, blk_off):
    n_tok = tt.shape[1]
    piece = gathered.shape[0]

    def body(*refs):
        if dst is None:
            _tc_body(None, *refs)
        else:
            _tc_body(*refs)

    dst_in_specs = [] if dst is None else [pl.BlockSpec((8, 128), lambda i: (0, 0))]
    dst_args = () if dst is None else (dst,)

    return pl.pallas_call(
        body,
        grid=(piece // BT,),
        in_specs=dst_in_specs + [
            pl.BlockSpec((BT, H), lambda i: (i, 0)),
            pl.BlockSpec((3, BT), lambda i, o=blk_off: (0, i + o)),
            pl.BlockSpec((3, H), lambda i: (0, 0)),
            pl.BlockSpec((1, H), lambda i: (0, 0)),
            pl.BlockSpec((1, H), lambda i: (0, 0)),
            pl.BlockSpec((1, H), lambda i: (0, 0)),
        ],
        out_specs=pl.BlockSpec((BT, H), lambda i, o=blk_off: (i + o, 0)),
        out_shape=jax.ShapeDtypeStruct((n_tok, H), jnp.float32),
        input_output_aliases={} if dst is None else {0: 0},
    )(*dst_args, gathered, tt, tt_w, tt_b, gamma, beta)


@jax.jit
def kernel(input_ids, token_type_ids, word_table, tt_w, tt_b, ln_gamma, ln_beta):
    b, s = input_ids.shape
    idx = input_ids.reshape(-1).astype(jnp.int32)
    tt = token_type_ids.reshape(-1, 3).T
    tt_b2 = tt_b.reshape(1, H)
    gam2 = ln_gamma.reshape(1, H)
    bet2 = ln_beta.reshape(1, H)

    out = None
    start = 0
    for piece in PIECES:
        gathered = _sc_gather(word_table, idx, start, piece)
        out = _tc_fused_piece(out, gathered, tt, tt_w, tt_b2, gam2, bet2,
                              start // BT)
        start += piece
    return out.reshape(b, s, H)
